# baseline (speedup 1.0000x reference)
import functools

import jax
import jax.numpy as jnp
from jax import lax
from jax.experimental import pallas as pl
from jax.experimental.pallas import tpu as pltpu
from jax.experimental.pallas import tpu_sc as plsc

N = 10000
E = 320000
H = 128
HH = H // 2
DE = 16
L3 = 3
TASKS = 112

NT = 16
K = 128
NCHC = E // K
NCH0 = NCHC // NT
NREM = NCHC - NCH0 * NT
NP = 10112
RPT = NP // NT

BE = 10000
NEB = E // BE


def _split8(a):
    return jnp.concatenate([a[:, :HH], a[:, HH:]], axis=0).reshape(2, 8, HH)


def _enc_body(x_ref, w_ref, b_ref, h_ref, hm_ref):
    h = jnp.dot(x_ref[...], w_ref[...], preferred_element_type=jnp.float32)
    h = h + b_ref[...]
    h_ref[...] = h
    hm_ref[...] = _split8(
        jnp.broadcast_to(jnp.max(h, axis=0, keepdims=True), (8, H)))


def _edges_body(ea_ref, w_ref, b_ref, e_ref, em_ref, acc_ref):
    i = pl.program_id(1)
    for ch in range(2):
        pe = jnp.dot(ea_ref[...], w_ref[0, ch],
                     preferred_element_type=jnp.float32)
        pe = pe + b_ref[0, ch]
        e_ref[0, ch] = pe
        bm = jnp.max(pe, axis=0, keepdims=True)
        bmh = jnp.maximum(bm[:, :HH], bm[:, HH:])

        @pl.when(i == 0)
        def _():
            acc_ref[ch] = jnp.broadcast_to(bmh, (8, HH))

        @pl.when(i > 0)
        def _():
            acc_ref[ch] = jnp.maximum(acc_ref[ch], bmh)

        @pl.when(i == NEB - 1)
        def _():
            em_ref[0, ch] = acc_ref[ch]


def _combine_body(res, s_ref, h_ref, w1_ref, b1_ref, g1_ref,
                  be1_ref, w2_ref, b2_ref, gn_ref, bn_ref,
                  hn_ref, hm_ref):
    s1 = jnp.concatenate([s_ref[0, :N, :HH], s_ref[1, :N, :HH]], axis=1)
    s2 = jnp.concatenate([s_ref[0, :N, HH:], s_ref[1, :N, HH:]], axis=1)
    h = h_ref[...]
    agg = s2 / (s1 + 1e-16)
    out = agg + h
    z = jnp.dot(out, w1_ref[...], preferred_element_type=jnp.float32)
    z = z + b1_ref[...]
    mu = jnp.mean(z, axis=0, keepdims=True)
    va = jnp.mean((z - mu) ** 2, axis=0, keepdims=True)
    z = (z - mu) / jnp.sqrt(va + 1e-5) * g1_ref[...] + be1_ref[...]
    z = jnp.maximum(z, 0.0)
    h1 = jnp.dot(z, w2_ref[...], preferred_element_type=jnp.float32)
    h1 = h1 + b2_ref[...]
    mu2 = jnp.mean(h1, axis=0, keepdims=True)
    va2 = jnp.mean((h1 - mu2) ** 2, axis=0, keepdims=True)
    h2 = (h1 - mu2) / jnp.sqrt(va2 + 1e-5) * gn_ref[...] + bn_ref[...]
    hn = jnp.maximum(h2, 0.0)
    if res:
        hn = hn + h
    hn_ref[...] = hn
    hm_ref[...] = _split8(
        jnp.broadcast_to(jnp.max(hn, axis=0, keepdims=True), (8, H)))


def _final_body(h1_ref, h2_ref, h3_ref, w_ref, b_ref, o_ref):
    cat = jnp.concatenate([h1_ref[...], h2_ref[...], h3_ref[...]], axis=1)
    o = jnp.dot(cat, w_ref[...], preferred_element_type=jnp.float32)
    o_ref[...] = o + b_ref[...]


def _encoder(x, enc_W, enc_b):
    return pl.pallas_call(
        _enc_body,
        out_shape=(
            jax.ShapeDtypeStruct((N, H), jnp.float32),
            jax.ShapeDtypeStruct((2, 8, HH), jnp.float32),
        ),
    )(x, enc_W, enc_b.reshape(1, H))


def _edges(edge_attr, leW, leb):
    ea2 = edge_attr.reshape(E // 2, 2 * DE)
    wh = jnp.stack([leW[:, :, :HH], leW[:, :, HH:]], axis=1)
    z = jnp.zeros_like(wh)
    w2 = jnp.concatenate([
        jnp.concatenate([wh, z], axis=-1),
        jnp.concatenate([z, wh], axis=-1),
    ], axis=2)
    bh = jnp.stack([leb[:, :HH], leb[:, HH:]], axis=1)
    b2 = jnp.concatenate([bh, bh], axis=-1)[:, :, None, :]
    return pl.pallas_call(
        _edges_body,
        grid=(L3, NEB),
        in_specs=[
            pl.BlockSpec((BE // 2, 2 * DE), lambda l, i: (i, 0)),
            pl.BlockSpec((1, 2, 2 * DE, H), lambda l, i: (l, 0, 0, 0)),
            pl.BlockSpec((1, 2, 1, H), lambda l, i: (l, 0, 0, 0)),
        ],
        out_specs=[
            pl.BlockSpec((1, 2, BE // 2, H), lambda l, i: (l, 0, i, 0)),
            pl.BlockSpec((1, 2, 8, HH), lambda l, i: (l, 0, 0, 0)),
        ],
        out_shape=(
            jax.ShapeDtypeStruct((L3, 2, E // 2, H), jnp.float32),
            jax.ShapeDtypeStruct((L3, 2, 8, HH), jnp.float32),
        ),
        scratch_shapes=[pltpu.VMEM((2, 8, HH), jnp.float32)],
    )(ea2, w2, b2)


def _combine(res, s, h, w1, b1, g1, be1, w2, b2, gn, bn):
    return pl.pallas_call(
        functools.partial(_combine_body, res),
        out_shape=(
            jax.ShapeDtypeStruct((N, H), jnp.float32),
            jax.ShapeDtypeStruct((2, 8, HH), jnp.float32),
        ),
    )(s, h, w1, b1.reshape(1, 2 * H), g1.reshape(1, 2 * H),
      be1.reshape(1, 2 * H), w2, b2.reshape(1, H), gn.reshape(1, H),
      bn.reshape(1, H))


def _final(h1, h2, h3, pred_W, pred_b):
    return pl.pallas_call(
        _final_body,
        out_shape=jax.ShapeDtypeStruct((N, TASKS), jnp.float32),
    )(h1, h2, h3, pred_W, pred_b.reshape(1, TASKS))


_mesh = plsc.VectorSubcoreMesh(core_axis_name="c", subcore_axis_name="s")


@functools.partial(
    pl.kernel,
    mesh=_mesh,
    out_type=jax.ShapeDtypeStruct((2, NP, H), jnp.float32),
    scratch_types=[
        pltpu.VMEM_SHARED((NP, H), jnp.float32),
        pltpu.VMEM((K,), jnp.int32),
        pltpu.VMEM((K,), jnp.int32),
        pltpu.VMEM((K,), jnp.int32),
        pltpu.VMEM((K,), jnp.int32),
        pltpu.VMEM((K, H), jnp.float32),
        pltpu.VMEM((K, H), jnp.float32),
        pltpu.VMEM((K // 2, H), jnp.float32),
        pltpu.VMEM((K // 2, H), jnp.float32),
        pltpu.VMEM((HH,), jnp.float32),
        pltpu.VMEM((HH,), jnp.float32),
        pltpu.SemaphoreType.DMA,
        pltpu.SemaphoreType.DMA,
        pltpu.SemaphoreType.DMA,
        pltpu.SemaphoreType.DMA,
        pltpu.SemaphoreType.DMA,
        pltpu.SemaphoreType.DMA,
        pltpu.SemaphoreType.DMA,
        pltpu.SemaphoreType.DMA,
    ],
)
def _sc_segment_softmax(h_tab, src, dst, e_l, hmax, emax, zrows, s_out,
                        acc, sr0, sr1, ds0, ds1, hr0, hr1, eb0, eb1,
                        bvec, tmpv,
                        si0, si1, sd0, sd1, sg0, sg1, se0, se1):
    c = lax.axis_index("c")
    s = lax.axis_index("s")
    choff = c * HH
    nch = NCH0 + jnp.where(s < NREM, 1, 0)
    srcb = (sr0, sr1)
    dstb = (ds0, ds1)
    hrows = (hr0, hr1)
    ebuf = (eb0, eb1)
    semi = (si0, si1)
    semd = (sd0, sd1)
    semg = (sg0, sg1)
    seme = (se0, se1)

    pltpu.sync_copy(hmax.at[c, 0], bvec)
    pltpu.sync_copy(emax.at[c, 0], tmpv)
    for v in range(HH // 16):
        sl = pl.ds(v * 16, 16)
        bvec[sl] = jnp.maximum(bvec[sl] + tmpv[sl], 0.0) + 1e-7

    pltpu.sync_copy(zrows, acc.at[pl.ds(s * RPT, RPT)])

    plsc.subcore_barrier()

    def _ebase(t):
        return (s + NT * t) * K

    def _issue_idx(t, b):
        sl = pl.ds(_ebase(t), K)
        pltpu.make_async_copy(src.at[sl], srcb[b], semi[b]).start()
        pltpu.make_async_copy(dst.at[sl], dstb[b], semd[b]).start()

    def _wait_idx(b):
        pltpu.make_async_copy(src.at[pl.ds(0, K)], srcb[b], semi[b]).wait()
        pltpu.make_async_copy(dst.at[pl.ds(0, K)], dstb[b], semd[b]).wait()

    def _issue_data(t, b):
        pltpu.make_async_copy(h_tab.at[srcb[b]], hrows[b], semg[b]).start()
        pltpu.make_async_copy(
            e_l.at[c, pl.ds((s + NT * t) * (K // 2), K // 2)], ebuf[b],
            seme[b]).start()

    def _wait_data(b):
        pltpu.make_async_copy(h_tab.at[srcb[b]], hrows[b], semg[b]).wait()
        pltpu.make_async_copy(
            e_l.at[c, pl.ds(0, K // 2)], ebuf[b], seme[b]).wait()

    bks = [bvec[pl.ds(v * 16, 16)] for v in range(HH // 16)]

    def _compute(b):
        hb = hrows[b]
        eb = ebuf[b]
        nv = HH // 16

        @plsc.parallel_loop(0, K // 2, unroll=8)
        def _rbody(r):
            for u in (0, 1):
                j = 2 * r + u
                hs = [hb[j, pl.ds(choff + v * 16, 16)] for v in range(nv)]
                es = [eb[r, pl.ds(u * HH + v * 16, 16)] for v in range(nv)]
                ms = [jnp.maximum(hs[v] + es[v], 0.0) + 1e-7
                      for v in range(nv)]
                exs = [jnp.exp(ms[v] - bks[v]) for v in range(nv)]
                for v in range(nv):
                    hb[j, pl.ds(v * 16, 16)] = exs[v]
                    hb[j, pl.ds(HH + v * 16, 16)] = ms[v] * exs[v]

    def _step(t, b):
        @pl.when(t + 1 < nch)
        def _():
            _wait_idx(1 - b)
            _issue_data(t + 1, 1 - b)

        _wait_data(b)
        _compute(b)
        pltpu.sync_copy(hrows[b], acc.at[dstb[b]], add=True)

        @pl.when(t + 2 < nch)
        def _():
            _issue_idx(t + 2, b)

    _issue_idx(0, 0)
    _wait_idx(0)
    _issue_data(0, 0)
    _issue_idx(1, 1)

    def _pair(i, carry):
        _step(2 * i, 0)
        _step(2 * i + 1, 1)
        return carry

    lax.fori_loop(0, NCH0 // 2, _pair, 0)

    @pl.when(nch > NCH0)
    def _():
        _step(NCH0, 0)

    plsc.subcore_barrier()

    pltpu.sync_copy(acc.at[pl.ds(s * RPT, RPT)],
                    s_out.at[c, pl.ds(s * RPT, RPT)])


def kernel(x, edge_index, edge_attr, enc_W, enc_b, lin_edge_W, lin_edge_b,
           mlp_W1, mlp_b1, mlp_bn_g, mlp_bn_b, mlp_W2, mlp_b2,
           norm_g, norm_b, pred_W, pred_b):
    h, hmax = _encoder(x, enc_W, enc_b)
    e_all, emax_all = _edges(edge_attr, lin_edge_W, lin_edge_b)

    zrows = jnp.zeros((RPT, H), jnp.float32)
    src = edge_index[0]
    dst = edge_index[1]
    inter = []
    for l in range(L3):
        s = _sc_segment_softmax(
            h, src, dst, e_all[l], hmax, emax_all[l], zrows)
        h, hmax = _combine(
            l > 0, s, h, mlp_W1[l], mlp_b1[l], mlp_bn_g[l],
            mlp_bn_b[l], mlp_W2[l], mlp_b2[l], norm_g[l], norm_b[l])
        inter.append(h)

    return _final(inter[0], inter[1], inter[2], pred_W, pred_b)


# speedup vs baseline: 1.6265x; 1.6265x over previous
import functools

import jax
import jax.numpy as jnp
from jax import lax
from jax.experimental import pallas as pl
from jax.experimental.pallas import tpu as pltpu
from jax.experimental.pallas import tpu_sc as plsc

N = 10000
E = 320000
H = 128
HH = H // 2
DE = 16
L3 = 3
TASKS = 112

NT = 16
K = 128
NCHC = E // K
NCH0 = NCHC // NT
NREM = NCHC - NCH0 * NT
NP = 10112
RPT = NP // NT

BE = 10000
NEB = E // BE


def _split8(a):
    return jnp.concatenate([a[:, :HH], a[:, HH:]], axis=0).reshape(2, 8, HH)


def _enc_body(x_ref, w_ref, b_ref, h_ref, hm_ref):
    h = jnp.dot(x_ref[...], w_ref[...], preferred_element_type=jnp.float32)
    h = h + b_ref[...]
    h_ref[...] = h
    hm_ref[...] = _split8(
        jnp.broadcast_to(jnp.max(h, axis=0, keepdims=True), (8, H)))


def _edges_body(ea_ref, w_ref, b_ref, e_ref, em_ref, acc_ref):
    i = pl.program_id(1)
    for ch in range(2):
        pe = jnp.dot(ea_ref[...], w_ref[0, ch],
                     preferred_element_type=jnp.float32)
        pe = pe + b_ref[0, ch]
        e_ref[0, ch] = pe
        bm = jnp.max(pe, axis=0, keepdims=True)
        bmh = jnp.maximum(bm[:, :HH], bm[:, HH:])

        @pl.when(i == 0)
        def _():
            acc_ref[ch] = jnp.broadcast_to(bmh, (8, HH))

        @pl.when(i > 0)
        def _():
            acc_ref[ch] = jnp.maximum(acc_ref[ch], bmh)

        @pl.when(i == NEB - 1)
        def _():
            em_ref[0, ch] = acc_ref[ch]


def _combine_body(res, s_ref, h_ref, w1_ref, b1_ref, g1_ref,
                  be1_ref, w2_ref, b2_ref, gn_ref, bn_ref,
                  hn_ref, hm_ref):
    s1 = jnp.concatenate([s_ref[0, :N, :HH], s_ref[1, :N, :HH]], axis=1)
    s2 = jnp.concatenate([s_ref[0, :N, HH:], s_ref[1, :N, HH:]], axis=1)
    h = h_ref[...]
    agg = s2 / (s1 + 1e-16)
    out = agg + h
    z = jnp.dot(out, w1_ref[...], preferred_element_type=jnp.float32)
    z = z + b1_ref[...]
    mu = jnp.mean(z, axis=0, keepdims=True)
    va = jnp.mean((z - mu) ** 2, axis=0, keepdims=True)
    z = (z - mu) / jnp.sqrt(va + 1e-5) * g1_ref[...] + be1_ref[...]
    z = jnp.maximum(z, 0.0)
    h1 = jnp.dot(z, w2_ref[...], preferred_element_type=jnp.float32)
    h1 = h1 + b2_ref[...]
    mu2 = jnp.mean(h1, axis=0, keepdims=True)
    va2 = jnp.mean((h1 - mu2) ** 2, axis=0, keepdims=True)
    h2 = (h1 - mu2) / jnp.sqrt(va2 + 1e-5) * gn_ref[...] + bn_ref[...]
    hn = jnp.maximum(h2, 0.0)
    if res:
        hn = hn + h
    hn_ref[...] = hn
    hm_ref[...] = _split8(
        jnp.broadcast_to(jnp.max(hn, axis=0, keepdims=True), (8, H)))


def _final_body(h1_ref, h2_ref, h3_ref, w_ref, b_ref, o_ref):
    cat = jnp.concatenate([h1_ref[...], h2_ref[...], h3_ref[...]], axis=1)
    o = jnp.dot(cat, w_ref[...], preferred_element_type=jnp.float32)
    o_ref[...] = o + b_ref[...]


def _encoder(x, enc_W, enc_b):
    return pl.pallas_call(
        _enc_body,
        out_shape=(
            jax.ShapeDtypeStruct((N, H), jnp.float32),
            jax.ShapeDtypeStruct((2, 8, HH), jnp.float32),
        ),
    )(x, enc_W, enc_b.reshape(1, H))


def _edges(edge_attr, leW, leb):
    ea2 = edge_attr.reshape(E // 2, 2 * DE)
    wh = jnp.stack([leW[:, :, :HH], leW[:, :, HH:]], axis=1)
    z = jnp.zeros_like(wh)
    w2 = jnp.concatenate([
        jnp.concatenate([wh, z], axis=-1),
        jnp.concatenate([z, wh], axis=-1),
    ], axis=2)
    bh = jnp.stack([leb[:, :HH], leb[:, HH:]], axis=1)
    b2 = jnp.concatenate([bh, bh], axis=-1)[:, :, None, :]
    outs = []
    for l in range(L3):
        outs.append(pl.pallas_call(
            _edges_body,
            grid=(1, NEB),
            in_specs=[
                pl.BlockSpec((BE // 2, 2 * DE), lambda ll, i: (i, 0)),
                pl.BlockSpec((1, 2, 2 * DE, H), lambda ll, i: (0, 0, 0, 0)),
                pl.BlockSpec((1, 2, 1, H), lambda ll, i: (0, 0, 0, 0)),
            ],
            out_specs=[
                pl.BlockSpec((1, 2, BE // 2, H), lambda ll, i: (0, 0, i, 0)),
                pl.BlockSpec((1, 2, 8, HH), lambda ll, i: (0, 0, 0, 0)),
            ],
            out_shape=(
                jax.ShapeDtypeStruct((1, 2, E // 2, H), jnp.float32),
                jax.ShapeDtypeStruct((1, 2, 8, HH), jnp.float32),
            ),
            scratch_shapes=[pltpu.VMEM((2, 8, HH), jnp.float32)],
        )(ea2, w2[l:l + 1], b2[l:l + 1]))
    e_all = [o[0][0] for o in outs]
    emax_all = [o[1][0] for o in outs]
    return e_all, emax_all


def _combine(res, s, h, w1, b1, g1, be1, w2, b2, gn, bn):
    return pl.pallas_call(
        functools.partial(_combine_body, res),
        out_shape=(
            jax.ShapeDtypeStruct((N, H), jnp.float32),
            jax.ShapeDtypeStruct((2, 8, HH), jnp.float32),
        ),
    )(s, h, w1, b1.reshape(1, 2 * H), g1.reshape(1, 2 * H),
      be1.reshape(1, 2 * H), w2, b2.reshape(1, H), gn.reshape(1, H),
      bn.reshape(1, H))


def _final(h1, h2, h3, pred_W, pred_b):
    return pl.pallas_call(
        _final_body,
        out_shape=jax.ShapeDtypeStruct((N, TASKS), jnp.float32),
    )(h1, h2, h3, pred_W, pred_b.reshape(1, TASKS))


_mesh = plsc.VectorSubcoreMesh(core_axis_name="c", subcore_axis_name="s")


@functools.partial(
    pl.kernel,
    mesh=_mesh,
    out_type=jax.ShapeDtypeStruct((2, NP, H), jnp.float32),
    scratch_types=[
        pltpu.VMEM_SHARED((NP, H), jnp.float32),
        pltpu.VMEM((K,), jnp.int32),
        pltpu.VMEM((K,), jnp.int32),
        pltpu.VMEM((K,), jnp.int32),
        pltpu.VMEM((K,), jnp.int32),
        pltpu.VMEM((K, H), jnp.float32),
        pltpu.VMEM((K, H), jnp.float32),
        pltpu.VMEM((K // 2, H), jnp.float32),
        pltpu.VMEM((K // 2, H), jnp.float32),
        pltpu.VMEM((HH,), jnp.float32),
        pltpu.VMEM((HH,), jnp.float32),
        pltpu.SemaphoreType.DMA,
        pltpu.SemaphoreType.DMA,
        pltpu.SemaphoreType.DMA,
        pltpu.SemaphoreType.DMA,
        pltpu.SemaphoreType.DMA,
        pltpu.SemaphoreType.DMA,
        pltpu.SemaphoreType.DMA,
        pltpu.SemaphoreType.DMA,
    ],
)
def _sc_segment_softmax(h_tab, src, dst, e_l, hmax, emax, zrows, s_out,
                        acc, sr0, sr1, ds0, ds1, hr0, hr1, eb0, eb1,
                        bvec, tmpv,
                        si0, si1, sd0, sd1, sg0, sg1, se0, se1):
    c = lax.axis_index("c")
    s = lax.axis_index("s")
    choff = c * HH
    nch = NCH0 + jnp.where(s < NREM, 1, 0)
    srcb = (sr0, sr1)
    dstb = (ds0, ds1)
    hrows = (hr0, hr1)
    ebuf = (eb0, eb1)
    semi = (si0, si1)
    semd = (sd0, sd1)
    semg = (sg0, sg1)
    seme = (se0, se1)

    pltpu.sync_copy(hmax.at[c, 0], bvec)
    pltpu.sync_copy(emax.at[c, 0], tmpv)
    for v in range(HH // 16):
        sl = pl.ds(v * 16, 16)
        bvec[sl] = jnp.maximum(bvec[sl] + tmpv[sl], 0.0) + 1e-7

    pltpu.sync_copy(zrows, acc.at[pl.ds(s * RPT, RPT)])

    plsc.subcore_barrier()

    def _ebase(t):
        return (s + NT * t) * K

    def _issue_idx(t, b):
        sl = pl.ds(_ebase(t), K)
        pltpu.make_async_copy(src.at[sl], srcb[b], semi[b]).start()
        pltpu.make_async_copy(dst.at[sl], dstb[b], semd[b]).start()

    def _wait_idx(b):
        pltpu.make_async_copy(src.at[pl.ds(0, K)], srcb[b], semi[b]).wait()
        pltpu.make_async_copy(dst.at[pl.ds(0, K)], dstb[b], semd[b]).wait()

    def _issue_data(t, b):
        pltpu.make_async_copy(h_tab.at[srcb[b]], hrows[b], semg[b]).start()
        pltpu.make_async_copy(
            e_l.at[c, pl.ds((s + NT * t) * (K // 2), K // 2)], ebuf[b],
            seme[b]).start()

    def _wait_data(b):
        pltpu.make_async_copy(h_tab.at[srcb[b]], hrows[b], semg[b]).wait()
        pltpu.make_async_copy(
            e_l.at[c, pl.ds(0, K // 2)], ebuf[b], seme[b]).wait()

    bks = [bvec[pl.ds(v * 16, 16)] for v in range(HH // 16)]

    def _compute(b):
        hb = hrows[b]
        eb = ebuf[b]
        nv = HH // 16

        @plsc.parallel_loop(0, K // 2, unroll=4)
        def _rbody(r):
            for u in (0, 1):
                j = 2 * r + u
                hs = [hb[j, pl.ds(choff + v * 16, 16)] for v in range(nv)]
                es = [eb[r, pl.ds(u * HH + v * 16, 16)] for v in range(nv)]
                ms = [jnp.maximum(hs[v] + es[v], 0.0) + 1e-7
                      for v in range(nv)]
                exs = [jnp.exp(ms[v] - bks[v]) for v in range(nv)]
                for v in range(nv):
                    hb[j, pl.ds(v * 16, 16)] = exs[v]
                    hb[j, pl.ds(HH + v * 16, 16)] = ms[v] * exs[v]

    def _step(t, b):
        @pl.when(t + 1 < nch)
        def _():
            _wait_idx(1 - b)
            _issue_data(t + 1, 1 - b)

        _wait_data(b)
        _compute(b)
        pltpu.sync_copy(hrows[b], acc.at[dstb[b]], add=True)

        @pl.when(t + 2 < nch)
        def _():
            _issue_idx(t + 2, b)

    _issue_idx(0, 0)
    _wait_idx(0)
    _issue_data(0, 0)
    _issue_idx(1, 1)

    def _pair(i, carry):
        _step(2 * i, 0)
        _step(2 * i + 1, 1)
        return carry

    lax.fori_loop(0, NCH0 // 2, _pair, 0)

    @pl.when(nch > NCH0)
    def _():
        _step(NCH0, 0)

    plsc.subcore_barrier()

    pltpu.sync_copy(acc.at[pl.ds(s * RPT, RPT)],
                    s_out.at[c, pl.ds(s * RPT, RPT)])


def kernel(x, edge_index, edge_attr, enc_W, enc_b, lin_edge_W, lin_edge_b,
           mlp_W1, mlp_b1, mlp_bn_g, mlp_bn_b, mlp_W2, mlp_b2,
           norm_g, norm_b, pred_W, pred_b):
    h, hmax = _encoder(x, enc_W, enc_b)
    e_all, emax_all = _edges(edge_attr, lin_edge_W, lin_edge_b)

    zrows = jnp.zeros((RPT, H), jnp.float32)
    src = edge_index[0]
    dst = edge_index[1]
    inter = []
    for l in range(L3):
        s = _sc_segment_softmax(
            h, src, dst, e_all[l], hmax, emax_all[l], zrows)
        h, hmax = _combine(
            l > 0, s, h, mlp_W1[l], mlp_b1[l], mlp_bn_g[l],
            mlp_bn_b[l], mlp_W2[l], mlp_b2[l], norm_g[l], norm_b[l])
        inter.append(h)

    return _final(inter[0], inter[1], inter[2], pred_W, pred_b)
